# baseline (speedup 1.0000x reference)
import functools

import jax
import jax.numpy as jnp
from jax import lax
from jax.experimental import pallas as pl
from jax.experimental.pallas import tpu as pltpu
from jax.experimental.pallas import tpu_sc as plsc

EMBED = 64
SEQ = 200
B = 4096
N = B * SEQ
NC = 2
NS = 16
NW = NC * NS
N_PER_W = N // NW
CHUNK = 128
N_CHUNKS = N_PER_W // CHUNK
NQ = EMBED // 16
NG = CHUNK // 16


def _make_kernel():
    mesh = plsc.VectorSubcoreMesh(core_axis_name="c", subcore_axis_name="s")

    @functools.partial(
        pl.kernel,
        mesh=mesh,
        out_type=jax.ShapeDtypeStruct((N, EMBED), jnp.float32),
        compiler_params=pltpu.CompilerParams(use_tc_tiling_on_sc=False),
        scratch_types=[
            pltpu.VMEM((3, CHUNK), jnp.int32),
            pltpu.VMEM((3, CHUNK), jnp.int32),
            pltpu.VMEM((3, CHUNK, EMBED), jnp.float32),
            pltpu.VMEM((2 * SEQ, EMBED), jnp.float32),
            pltpu.VMEM((2, EMBED), jnp.float32),
            pltpu.SemaphoreType.DMA,
            pltpu.SemaphoreType.DMA,
            pltpu.SemaphoreType.DMA,
            pltpu.SemaphoreType.DMA,
        ],
    )
    def k(tok_hbm, seg_hbm, table_hbm, segtab_hbm, pos_hbm, out_hbm,
          tok_v, seg_v, rows_v, fused_v, st_v, sem_it, sem_is, sem_g, sem_w):
        wid = lax.axis_index("s") * NC + lax.axis_index("c")
        base = wid * N_PER_W

        pltpu.sync_copy(segtab_hbm, st_v)
        pltpu.sync_copy(pos_hbm, fused_v.at[pl.ds(0, SEQ)])
        pltpu.sync_copy(pos_hbm, fused_v.at[pl.ds(SEQ, SEQ)])

        def fuse_body(l, _):
            for q in range(NQ):
                sl = pl.ds(q * 16, 16)
                fused_v[l, sl] = fused_v[l, sl] + st_v[0, sl]
                fused_v[SEQ + l, sl] = fused_v[SEQ + l, sl] + st_v[1, sl]
            return 0

        lax.fori_loop(0, SEQ, fuse_body, 0)

        def stage_idx(c):
            buf = lax.rem(c, 3)
            cbase = base + c * CHUNK
            pltpu.async_copy(tok_hbm.at[pl.ds(cbase, CHUNK)], tok_v.at[buf],
                             sem_it)
            pltpu.async_copy(seg_hbm.at[pl.ds(cbase, CHUNK)], seg_v.at[buf],
                             sem_is)

        def wait_idx(c):
            buf = lax.rem(c, 3)
            cbase = base + c * CHUNK
            pltpu.make_async_copy(tok_hbm.at[pl.ds(cbase, CHUNK)],
                                  tok_v.at[buf], sem_it).wait()
            pltpu.make_async_copy(seg_hbm.at[pl.ds(cbase, CHUNK)],
                                  seg_v.at[buf], sem_is).wait()

        def start_gather(c):
            buf = lax.rem(c, 3)
            pltpu.async_copy(table_hbm.at[tok_v.at[buf]], rows_v.at[buf],
                             sem_g)

        def wait_gather(c):
            buf = lax.rem(c, 3)
            pltpu.make_async_copy(table_hbm.at[tok_v.at[buf]],
                                  rows_v.at[buf], sem_g).wait()

        def start_write(c):
            buf = lax.rem(c, 3)
            cbase = base + c * CHUNK
            pltpu.async_copy(rows_v.at[buf],
                             out_hbm.at[pl.ds(cbase, CHUNK)], sem_w)

        def wait_write(c):
            buf = lax.rem(c, 3)
            cbase = base + c * CHUNK
            pltpu.make_async_copy(rows_v.at[buf],
                                  out_hbm.at[pl.ds(cbase, CHUNK)],
                                  sem_w).wait()

        def chunk_body(c, _):
            buf = lax.rem(c, 3)

            @pl.when(c + 2 < N_CHUNKS)
            def _():
                stage_idx(c + 2)

            @pl.when(c + 1 < N_CHUNKS)
            def _():
                @pl.when(c >= 2)
                def _():
                    wait_write(c - 2)

                wait_idx(c + 1)
                start_gather(c + 1)

            l0 = lax.rem(c * CHUNK, SEQ)
            wait_gather(c)

            for g in range(NG):
                g16 = g * 16
                l = l0 + g16 + lax.iota(jnp.int32, 16)
                l = jnp.where(l >= SEQ, l - SEQ, l)
                fidx = seg_v[buf, pl.ds(g16, 16)] * SEQ + l
                for i in range(16):
                    r = fidx[i]
                    row = g16 + i
                    for q in range(NQ):
                        sl = pl.ds(q * 16, 16)
                        rows_v[buf, row, sl] = (rows_v[buf, row, sl]
                                                + fused_v[r, sl])
            start_write(c)
            return 0

        stage_idx(0)
        stage_idx(1)
        wait_idx(0)
        start_gather(0)
        lax.fori_loop(0, N_CHUNKS, chunk_body, 0)
        wait_write(N_CHUNKS - 2)
        wait_write(N_CHUNKS - 1)

    return k


_sc_kernel = _make_kernel()


def kernel(tokens, segments, token_table, segment_table, pos_encoding):
    b, l = tokens.shape
    tok = tokens.reshape(-1).astype(jnp.int32)
    seg = segments.reshape(-1).astype(jnp.int32)
    pos = pos_encoding[0, :l]
    out = _sc_kernel(tok, seg, token_table, segment_table, pos)
    return out.reshape(b, l, EMBED)


# speedup vs baseline: 1.1349x; 1.1349x over previous
import functools

import jax
import jax.numpy as jnp
from jax import lax
from jax.experimental import pallas as pl
from jax.experimental.pallas import tpu as pltpu
from jax.experimental.pallas import tpu_sc as plsc

EMBED = 64
SEQ = 200
B = 4096
N = B * SEQ
NC = 2
NS = 16
NW = NC * NS
N_PER_W = N // NW
CHUNK = 128
N_CHUNKS = N_PER_W // CHUNK
NQ = EMBED // 16
NG = CHUNK // 16


def _make_kernel():
    mesh = plsc.VectorSubcoreMesh(core_axis_name="c", subcore_axis_name="s")

    @functools.partial(
        pl.kernel,
        mesh=mesh,
        out_type=jax.ShapeDtypeStruct((N, EMBED), jnp.float32),
        compiler_params=pltpu.CompilerParams(use_tc_tiling_on_sc=False),
        scratch_types=[
            pltpu.VMEM((4, CHUNK), jnp.int32),
            pltpu.VMEM((4, CHUNK), jnp.int32),
            pltpu.VMEM((4, CHUNK, EMBED), jnp.float32),
            pltpu.VMEM((2 * SEQ, EMBED), jnp.float32),
            pltpu.VMEM((2, EMBED), jnp.float32),
            pltpu.SemaphoreType.DMA,
            pltpu.SemaphoreType.DMA,
            pltpu.SemaphoreType.DMA,
            pltpu.SemaphoreType.DMA,
        ],
    )
    def k(tok_hbm, seg_hbm, table_hbm, segtab_hbm, pos_hbm, out_hbm,
          tok_v, seg_v, rows_v, fused_v, st_v, sem_it, sem_is, sem_g, sem_w):
        wid = lax.axis_index("s") * NC + lax.axis_index("c")
        base = wid * N_PER_W

        pltpu.sync_copy(segtab_hbm, st_v)
        pltpu.sync_copy(pos_hbm, fused_v.at[pl.ds(0, SEQ)])
        pltpu.sync_copy(pos_hbm, fused_v.at[pl.ds(SEQ, SEQ)])

        def fuse_body(l, _):
            for q in range(NQ):
                sl = pl.ds(q * 16, 16)
                fused_v[l, sl] = fused_v[l, sl] + st_v[0, sl]
                fused_v[SEQ + l, sl] = fused_v[SEQ + l, sl] + st_v[1, sl]
            return 0

        lax.fori_loop(0, SEQ, fuse_body, 0)

        def stage_idx(c):
            buf = lax.rem(c, 4)
            cbase = base + c * CHUNK
            pltpu.async_copy(tok_hbm.at[pl.ds(cbase, CHUNK)], tok_v.at[buf],
                             sem_it)
            pltpu.async_copy(seg_hbm.at[pl.ds(cbase, CHUNK)], seg_v.at[buf],
                             sem_is)

        def wait_idx(c):
            buf = lax.rem(c, 4)
            cbase = base + c * CHUNK
            pltpu.make_async_copy(tok_hbm.at[pl.ds(cbase, CHUNK)],
                                  tok_v.at[buf], sem_it).wait()
            pltpu.make_async_copy(seg_hbm.at[pl.ds(cbase, CHUNK)],
                                  seg_v.at[buf], sem_is).wait()

        def start_gather(c):
            buf = lax.rem(c, 4)
            pltpu.async_copy(table_hbm.at[tok_v.at[buf]], rows_v.at[buf],
                             sem_g)

        def wait_gather(c):
            buf = lax.rem(c, 4)
            pltpu.make_async_copy(table_hbm.at[tok_v.at[buf]],
                                  rows_v.at[buf], sem_g).wait()

        def start_write(c):
            buf = lax.rem(c, 4)
            cbase = base + c * CHUNK
            pltpu.async_copy(rows_v.at[buf],
                             out_hbm.at[pl.ds(cbase, CHUNK)], sem_w)

        def wait_write(c):
            buf = lax.rem(c, 4)
            cbase = base + c * CHUNK
            pltpu.make_async_copy(rows_v.at[buf],
                                  out_hbm.at[pl.ds(cbase, CHUNK)],
                                  sem_w).wait()

        def chunk_body(c, _):
            buf = lax.rem(c, 4)

            @pl.when(c + 3 < N_CHUNKS)
            def _():
                stage_idx(c + 3)

            @pl.when(c >= 2)
            def _():
                wait_write(c - 2)

            @pl.when(c + 2 < N_CHUNKS)
            def _():
                wait_idx(c + 2)
                start_gather(c + 2)

            l0 = lax.rem(c * CHUNK, SEQ)
            wait_gather(c)

            def add_group(g, _):
                g16 = g * 16
                l = l0 + g16 + lax.iota(jnp.int32, 16)
                l = jnp.where(l >= SEQ, l - SEQ, l)
                fidx = seg_v[buf, pl.ds(g16, 16)] * SEQ + l
                for i in range(16):
                    r = fidx[i]
                    row = g16 + i
                    for q in range(NQ):
                        sl = pl.ds(q * 16, 16)
                        rows_v[buf, row, sl] = (rows_v[buf, row, sl]
                                                + fused_v[r, sl])
                return 0

            lax.fori_loop(0, NG, add_group, 0)
            start_write(c)
            return 0

        stage_idx(0)
        stage_idx(1)
        stage_idx(2)
        wait_idx(0)
        start_gather(0)
        wait_idx(1)
        start_gather(1)
        lax.fori_loop(0, N_CHUNKS, chunk_body, 0)
        wait_write(N_CHUNKS - 2)
        wait_write(N_CHUNKS - 1)

    return k


_sc_kernel = _make_kernel()


def kernel(tokens, segments, token_table, segment_table, pos_encoding):
    b, l = tokens.shape
    tok = tokens.reshape(-1).astype(jnp.int32)
    seg = segments.reshape(-1).astype(jnp.int32)
    pos = pos_encoding[0, :l]
    out = _sc_kernel(tok, seg, token_table, segment_table, pos)
    return out.reshape(b, l, EMBED)
